# Initial kernel scaffold; baseline (speedup 1.0000x reference)
import jax
import jax.numpy as jnp
from jax.experimental import pallas as pl


def kernel(edge_index, weight_vector, id_embedding):
    raise NotImplementedError("write your pallas kernel here")


# speedup vs baseline: 8.6519x; 8.6519x over previous
import functools

import jax
import jax.numpy as jnp
from jax import lax
from jax.experimental import pallas as pl
from jax.experimental.pallas import tpu as pltpu
from jax.experimental.pallas import tpu_sc as plsc

N = 10000
D = 128
NC = 2
NS = 16
L = 16
NW = NC * NS
N_PAD = 10240
ROWS_PER_TILE = N_PAD // NS
CHUNK = 128
E2 = 640000
BCH = 8
NBLK = 20
CH_PER_W = NBLK * BCH
E_PAD = NW * CH_PER_W * CHUNK
ROW_BLK = 256


def _norm_body(x_ref, o_ref):
    x = x_ref[...]
    nrm = jnp.sqrt(jnp.sum(x * x, axis=1, keepdims=True))
    o_ref[...] = x / jnp.maximum(nrm, 1e-12)


def _l2norm(x_pad):
    return pl.pallas_call(
        _norm_body,
        out_shape=jax.ShapeDtypeStruct((N_PAD, D), jnp.float32),
        grid=(N_PAD // ROW_BLK,),
        in_specs=[pl.BlockSpec((ROW_BLK, D), lambda i: (i, 0))],
        out_specs=pl.BlockSpec((ROW_BLK, D), lambda i: (i, 0)),
    )(x_pad)


_GATHER_DNUMS = lax.GatherDimensionNumbers(
    offset_dims=(), collapsed_slice_dims=(0,), start_index_map=(0,))


def _dyn_splat(vec, lane):
    idx = jnp.full((L, 1), lane, jnp.int32)
    return lax.gather(vec, idx, _GATHER_DNUMS, slice_sizes=(1,),
                      mode=lax.GatherScatterMode.PROMISE_IN_BOUNDS)


def _make_layer(with_cnt):
    mesh = plsc.VectorSubcoreMesh(core_axis_name="c", subcore_axis_name="s")
    out_type = [jax.ShapeDtypeStruct((NC, N_PAD, D), jnp.float32)]
    if with_cnt:
        out_type.append(jax.ShapeDtypeStruct((NW, N_PAD), jnp.float32))
    scratch = [
        pltpu.VMEM_SHARED((N_PAD, D), jnp.float32),
        pltpu.VMEM((CHUNK, D), jnp.float32),
        pltpu.VMEM((BCH, CHUNK), jnp.int32),
        pltpu.VMEM((BCH, CHUNK), jnp.int32),
        pltpu.VMEM((BCH, CHUNK), jnp.float32),
        pltpu.VMEM((N_PAD,), jnp.float32),
        pltpu.SemaphoreType.DMA,
    ]

    def body(src_hbm, dst_hbm, w_hbm, x_hbm, *rest):
        if with_cnt:
            part_hbm, cnt_hbm, accum, rows, srcv, dstv, wv, cntv, sem = rest
        else:
            part_hbm, accum, rows, srcv, dstv, wv, cntv, sem = rest
        c = lax.axis_index("c")
        s = lax.axis_index("s")
        wid = s * NC + c
        base = s * ROWS_PER_TILE
        zero = jnp.zeros((L,), jnp.float32)

        def zr(r, carry):
            for k in range(D // L):
                rows[r, pl.ds(k * L, L)] = zero
            return carry
        lax.fori_loop(0, CHUNK, zr, 0)
        for t in range(ROWS_PER_TILE // CHUNK):
            pltpu.sync_copy(rows, accum.at[pl.ds(base + t * CHUNK, CHUNK)])

        if with_cnt:
            def zc(i, carry):
                cntv[pl.ds(i * L, L)] = zero
                return carry
            lax.fori_loop(0, N_PAD // L, zc, 0)

        plsc.subcore_barrier()

        iota = lax.iota(jnp.int32, L)
        ones = jnp.ones((L,), jnp.float32)

        def block_body(b, carry):
            pltpu.sync_copy(src_hbm.at[wid, pl.ds(b * BCH, BCH)], srcv)
            pltpu.sync_copy(dst_hbm.at[wid, pl.ds(b * BCH, BCH)], dstv)
            pltpu.sync_copy(w_hbm.at[wid, pl.ds(b * BCH, BCH)], wv)

            def chunk_body(j, carry1):
                pltpu.async_copy(x_hbm.at[srcv.at[j]], rows, sem).wait()

                def group_body(g, carry2):
                    wvec = wv[j, pl.ds(g * L, L)]
                    for lane in range(L):
                        wspl = _dyn_splat(wvec, lane)
                        e = g * L + lane
                        for k in range(D // L):
                            rows[e, pl.ds(k * L, L)] = (
                                rows[e, pl.ds(k * L, L)] * wspl)
                    if with_cnt:
                        dvec = dstv[j, pl.ds(g * L, L)]
                        for lane in range(L):
                            plsc.addupdate_scatter(
                                cntv, [dvec], ones, mask=iota == lane)
                    return carry2
                lax.fori_loop(0, CHUNK // L, group_body, 0)

                pltpu.sync_copy(rows, accum.at[dstv.at[j]], add=True)
                return carry1
            lax.fori_loop(0, BCH, chunk_body, 0)
            return carry
        lax.fori_loop(0, NBLK, block_body, 0)

        plsc.subcore_barrier()

        for t in range(ROWS_PER_TILE // CHUNK):
            r0 = base + t * CHUNK
            pltpu.sync_copy(accum.at[pl.ds(r0, CHUNK)], rows)
            pltpu.sync_copy(rows, part_hbm.at[c, pl.ds(r0, CHUNK)])
        if with_cnt:
            pltpu.sync_copy(cntv, cnt_hbm.at[wid])

    return pl.kernel(
        body, out_type=tuple(out_type), mesh=mesh, scratch_types=scratch,
        compiler_params=pltpu.CompilerParams(needs_layout_passes=False))


_layer1 = _make_layer(True)
_layer2 = _make_layer(False)


def _leaky(y):
    return jnp.where(y >= 0, y, 0.01 * y)


def _combine1_body(p_ref, c_ref, o_ref):
    acc = p_ref[0] + p_ref[1]
    cnt = jnp.sum(c_ref[...], axis=0)
    o_ref[...] = _leaky(acc / jnp.maximum(cnt, 1.0)[:, None])


def _combine1(part, cnt32):
    return pl.pallas_call(
        _combine1_body,
        out_shape=jax.ShapeDtypeStruct((N_PAD, D), jnp.float32),
        grid=(N_PAD // ROW_BLK,),
        in_specs=[
            pl.BlockSpec((NC, ROW_BLK, D), lambda i: (0, i, 0)),
            pl.BlockSpec((NW, ROW_BLK), lambda i: (0, i)),
        ],
        out_specs=pl.BlockSpec((ROW_BLK, D), lambda i: (i, 0)),
    )(part, cnt32)


def _combine2_body(p_ref, c_ref, x_ref, x1_ref, o_ref):
    acc = p_ref[0] + p_ref[1]
    cnt = jnp.sum(c_ref[...], axis=0)
    x2 = _leaky(acc / jnp.maximum(cnt, 1.0)[:, None])
    o_ref[...] = x_ref[...] + x1_ref[...] + x2


def _combine2(part, cnt32, x, x1):
    return pl.pallas_call(
        _combine2_body,
        out_shape=jax.ShapeDtypeStruct((N_PAD, D), jnp.float32),
        grid=(N_PAD // ROW_BLK,),
        in_specs=[
            pl.BlockSpec((NC, ROW_BLK, D), lambda i: (0, i, 0)),
            pl.BlockSpec((NW, ROW_BLK), lambda i: (0, i)),
            pl.BlockSpec((ROW_BLK, D), lambda i: (i, 0)),
            pl.BlockSpec((ROW_BLK, D), lambda i: (i, 0)),
        ],
        out_specs=pl.BlockSpec((ROW_BLK, D), lambda i: (i, 0)),
    )(part, cnt32, x, x1)


def kernel(edge_index, weight_vector, id_embedding):
    src = jnp.concatenate([edge_index[0], edge_index[1]])
    dst = jnp.concatenate([edge_index[1], edge_index[0]])
    w = weight_vector[:, 0]

    pad = E_PAD - E2
    ar = jnp.arange(pad, dtype=jnp.int32)
    src_p = jnp.concatenate([src, (ar * 97) % N_PAD])
    dst_p = jnp.concatenate([dst, N + (ar % (N_PAD - N))])
    w_p = jnp.concatenate([w, jnp.zeros((pad,), jnp.float32)])
    src_r = src_p.reshape(NW, CH_PER_W, CHUNK)
    dst_r = dst_p.reshape(NW, CH_PER_W, CHUNK)
    w_r = w_p.reshape(NW, CH_PER_W, CHUNK)

    x_pad = jnp.pad(id_embedding, ((0, N_PAD - N), (0, 0)))
    x = _l2norm(x_pad)

    part1, cnt32 = _layer1(src_r, dst_r, w_r, x)
    x1 = _combine1(part1, cnt32)

    (part2,) = _layer2(src_r, dst_r, w_r, x1)
    out = _combine2(part2, cnt32, x, x1)
    return out[:N]


# speedup vs baseline: 11.7551x; 1.3587x over previous
import jax
import jax.numpy as jnp
from jax import lax
from jax.experimental import pallas as pl
from jax.experimental.pallas import tpu as pltpu
from jax.experimental.pallas import tpu_sc as plsc

N = 10000
D = 128
NC = 2
NS = 16
L = 16
NW = NC * NS
N_PAD = 10112
ROWS_PER_TILE = N_PAD // NS
CHUNK = 128
E2 = 640000
BCH = 16
NBLK = 10
CH_PER_W = NBLK * BCH
E_PAD = NW * CH_PER_W * CHUNK
ROW_BLK = 128
_EPI = [CHUNK] * (ROWS_PER_TILE // CHUNK) + (
    [ROWS_PER_TILE % CHUNK] if ROWS_PER_TILE % CHUNK else [])


def _norm_body(x_ref, o_ref):
    x = x_ref[...]
    nrm = jnp.sqrt(jnp.sum(x * x, axis=1, keepdims=True))
    o_ref[...] = x / jnp.maximum(nrm, 1e-12)


def _l2norm(x_pad):
    return pl.pallas_call(
        _norm_body,
        out_shape=jax.ShapeDtypeStruct((N_PAD, D), jnp.float32),
        grid=(N_PAD // ROW_BLK,),
        in_specs=[pl.BlockSpec((ROW_BLK, D), lambda i: (i, 0))],
        out_specs=pl.BlockSpec((ROW_BLK, D), lambda i: (i, 0)),
    )(x_pad)


_GATHER_DNUMS = lax.GatherDimensionNumbers(
    offset_dims=(), collapsed_slice_dims=(0,), start_index_map=(0,))


def _dyn_splat(vec, lane):
    idx = jnp.full((L, 1), lane, jnp.int32)
    return lax.gather(vec, idx, _GATHER_DNUMS, slice_sizes=(1,),
                      mode=lax.GatherScatterMode.PROMISE_IN_BOUNDS)


def _make_layer(with_cnt):
    mesh = plsc.VectorSubcoreMesh(core_axis_name="c", subcore_axis_name="s")
    out_type = [jax.ShapeDtypeStruct((NC, N_PAD, D), jnp.float32)]
    if with_cnt:
        out_type.append(jax.ShapeDtypeStruct((NW, N_PAD), jnp.float32))
    scratch = [
        pltpu.VMEM_SHARED((N_PAD, D), jnp.float32),
        pltpu.VMEM((CHUNK, D), jnp.float32),
        pltpu.VMEM((CHUNK, D), jnp.float32),
        pltpu.VMEM((BCH, CHUNK), jnp.int32),
        pltpu.VMEM((BCH, CHUNK), jnp.int32),
        pltpu.VMEM((BCH, CHUNK), jnp.float32),
        pltpu.SemaphoreType.DMA,
        pltpu.SemaphoreType.DMA,
        pltpu.SemaphoreType.DMA,
        pltpu.SemaphoreType.DMA,
    ]
    if with_cnt:
        scratch.append(pltpu.VMEM((N_PAD,), jnp.float32))

    def body(src_hbm, dst_hbm, w_hbm, x_hbm, *rest):
        if with_cnt:
            (part_hbm, cnt_hbm, accum, rows0, rows1, srcv, dstv, wv,
             g0, g1, s0, s1, cntv) = rest
        else:
            (part_hbm, accum, rows0, rows1, srcv, dstv, wv,
             g0, g1, s0, s1) = rest
            cnt_hbm = cntv = None
        c = lax.axis_index("c")
        s = lax.axis_index("s")
        wid = s * NC + c
        base = s * ROWS_PER_TILE
        zero = jnp.zeros((L,), jnp.float32)

        def zr(r, carry):
            for k in range(D // L):
                rows0[r, pl.ds(k * L, L)] = zero
            return carry
        lax.fori_loop(0, CHUNK, zr, 0)
        r0 = base
        for ln in _EPI:
            pltpu.sync_copy(rows0.at[pl.ds(0, ln)], accum.at[pl.ds(r0, ln)])
            r0 += ln

        if with_cnt:
            def zc(i, carry):
                cntv[pl.ds(i * L, L)] = zero
                return carry
            lax.fori_loop(0, N_PAD // L, zc, 0)

        plsc.subcore_barrier()

        iota = lax.iota(jnp.int32, L)
        ones = jnp.ones((L,), jnp.float32)

        def issue_gather(jrow, buf, sem):
            pltpu.async_copy(x_hbm.at[srcv.at[jrow]], buf, sem)

        def wait_gather(buf, sem):
            pltpu.make_async_copy(x_hbm.at[srcv.at[0]], buf, sem).wait()

        def issue_scatter(jrow, buf, sem):
            pltpu.async_copy(buf, accum.at[dstv.at[jrow]], sem, add=True)

        def wait_scatter(buf, sem):
            pltpu.make_async_copy(buf, accum.at[dstv.at[0]], sem).wait()

        def multiply(buf, jrow):
            def group_body(g, carry):
                wvec = wv[jrow, pl.ds(g * L, L)]
                for lane in range(L):
                    wspl = _dyn_splat(wvec, lane)
                    e = g * L + lane
                    for k in range(D // L):
                        buf[e, pl.ds(k * L, L)] = (
                            buf[e, pl.ds(k * L, L)] * wspl)
                if with_cnt:
                    dvec = dstv[jrow, pl.ds(g * L, L)]
                    for lane in range(L):
                        plsc.addupdate_scatter(
                            cntv, [dvec], ones, mask=iota == lane)
                return carry
            lax.fori_loop(0, CHUNK // L, group_body, 0)

        def block_body(b, carry):
            pltpu.sync_copy(src_hbm.at[wid, pl.ds(b * BCH, BCH)], srcv)
            pltpu.sync_copy(dst_hbm.at[wid, pl.ds(b * BCH, BCH)], dstv)
            pltpu.sync_copy(w_hbm.at[wid, pl.ds(b * BCH, BCH)], wv)
            issue_gather(0, rows0, g0)

            def pair_body(t, carry1):
                je = 2 * t
                jo = 2 * t + 1

                @pl.when(t > 0)
                def _():
                    wait_scatter(rows1, s1)
                issue_gather(jo, rows1, g1)
                wait_gather(rows0, g0)
                multiply(rows0, je)
                issue_scatter(je, rows0, s0)

                wait_gather(rows1, g1)
                multiply(rows1, jo)
                issue_scatter(jo, rows1, s1)
                wait_scatter(rows0, s0)

                @pl.when(t < BCH // 2 - 1)
                def _():
                    issue_gather(je + 2, rows0, g0)
                return carry1
            lax.fori_loop(0, BCH // 2, pair_body, 0)
            wait_scatter(rows1, s1)
            return carry
        lax.fori_loop(0, NBLK, block_body, 0)

        plsc.subcore_barrier()

        r0 = base
        for ln in _EPI:
            pltpu.sync_copy(accum.at[pl.ds(r0, ln)], rows0.at[pl.ds(0, ln)])
            pltpu.sync_copy(rows0.at[pl.ds(0, ln)],
                            part_hbm.at[c, pl.ds(r0, ln)])
            r0 += ln
        if with_cnt:
            pltpu.sync_copy(cntv, cnt_hbm.at[wid])

    return pl.kernel(
        body, out_type=tuple(out_type), mesh=mesh, scratch_types=scratch,
        compiler_params=pltpu.CompilerParams(needs_layout_passes=False))


_layer1 = _make_layer(True)
_layer2 = _make_layer(False)


def _leaky(y):
    return jnp.where(y >= 0, y, 0.01 * y)


def _combine1_body(p_ref, c_ref, o_ref):
    acc = p_ref[0] + p_ref[1]
    cnt = jnp.sum(c_ref[...], axis=0)
    o_ref[...] = _leaky(acc / jnp.maximum(cnt, 1.0)[:, None])


def _combine1(part, cnt32):
    return pl.pallas_call(
        _combine1_body,
        out_shape=jax.ShapeDtypeStruct((N_PAD, D), jnp.float32),
        grid=(N_PAD // ROW_BLK,),
        in_specs=[
            pl.BlockSpec((NC, ROW_BLK, D), lambda i: (0, i, 0)),
            pl.BlockSpec((NW, ROW_BLK), lambda i: (0, i)),
        ],
        out_specs=pl.BlockSpec((ROW_BLK, D), lambda i: (i, 0)),
    )(part, cnt32)


def _combine2_body(p_ref, c_ref, x_ref, x1_ref, o_ref):
    acc = p_ref[0] + p_ref[1]
    cnt = jnp.sum(c_ref[...], axis=0)
    x2 = _leaky(acc / jnp.maximum(cnt, 1.0)[:, None])
    o_ref[...] = x_ref[...] + x1_ref[...] + x2


def _combine2(part, cnt32, x, x1):
    return pl.pallas_call(
        _combine2_body,
        out_shape=jax.ShapeDtypeStruct((N_PAD, D), jnp.float32),
        grid=(N_PAD // ROW_BLK,),
        in_specs=[
            pl.BlockSpec((NC, ROW_BLK, D), lambda i: (0, i, 0)),
            pl.BlockSpec((NW, ROW_BLK), lambda i: (0, i)),
            pl.BlockSpec((ROW_BLK, D), lambda i: (i, 0)),
            pl.BlockSpec((ROW_BLK, D), lambda i: (i, 0)),
        ],
        out_specs=pl.BlockSpec((ROW_BLK, D), lambda i: (i, 0)),
    )(part, cnt32, x, x1)


def kernel(edge_index, weight_vector, id_embedding):
    src = jnp.concatenate([edge_index[0], edge_index[1]])
    dst = jnp.concatenate([edge_index[1], edge_index[0]])
    w = weight_vector[:, 0]

    pad = E_PAD - E2
    ar = jnp.arange(pad, dtype=jnp.int32)
    src_p = jnp.concatenate([src, (ar * 97) % N_PAD])
    dst_p = jnp.concatenate([dst, N + (ar % (N_PAD - N))])
    w_p = jnp.concatenate([w, jnp.zeros((pad,), jnp.float32)])
    src_r = src_p.reshape(NW, CH_PER_W, CHUNK)
    dst_r = dst_p.reshape(NW, CH_PER_W, CHUNK)
    w_r = w_p.reshape(NW, CH_PER_W, CHUNK)

    x_pad = jnp.pad(id_embedding, ((0, N_PAD - N), (0, 0)))
    x = _l2norm(x_pad)

    part1, cnt32 = _layer1(src_r, dst_r, w_r, x)
    x1 = _combine1(part1, cnt32)

    (part2,) = _layer2(src_r, dst_r, w_r, x1)
    out = _combine2(part2, cnt32, x, x1)
    return out[:N]


# speedup vs baseline: 13.5648x; 1.1540x over previous
import jax
import jax.numpy as jnp
from jax import lax
from jax.experimental import pallas as pl
from jax.experimental.pallas import tpu as pltpu
from jax.experimental.pallas import tpu_sc as plsc

N = 10000
D = 128
NC = 2
NS = 16
L = 16
NW = NC * NS
N_PAD = 10112
ROWS_PER_TILE = N_PAD // NS
CHUNK = 64
E2 = 640000
BCH = 32
NBLK = 10
CH_PER_W = NBLK * BCH
E_PAD = NW * CH_PER_W * CHUNK
ROW_BLK = 128
_EPI = [CHUNK] * (ROWS_PER_TILE // CHUNK) + (
    [ROWS_PER_TILE % CHUNK] if ROWS_PER_TILE % CHUNK else [])


def _norm_body(x_ref, o_ref):
    x = x_ref[...]
    nrm = jnp.sqrt(jnp.sum(x * x, axis=1, keepdims=True))
    o_ref[...] = x / jnp.maximum(nrm, 1e-12)


def _l2norm(x_pad):
    return pl.pallas_call(
        _norm_body,
        out_shape=jax.ShapeDtypeStruct((N_PAD, D), jnp.float32),
        grid=(N_PAD // ROW_BLK,),
        in_specs=[pl.BlockSpec((ROW_BLK, D), lambda i: (i, 0))],
        out_specs=pl.BlockSpec((ROW_BLK, D), lambda i: (i, 0)),
    )(x_pad)


_GATHER_DNUMS = lax.GatherDimensionNumbers(
    offset_dims=(), collapsed_slice_dims=(0,), start_index_map=(0,))


def _dyn_splat(vec, lane):
    idx = jnp.full((L, 1), lane, jnp.int32)
    return lax.gather(vec, idx, _GATHER_DNUMS, slice_sizes=(1,),
                      mode=lax.GatherScatterMode.PROMISE_IN_BOUNDS)


def _make_layer(with_cnt):
    mesh = plsc.VectorSubcoreMesh(core_axis_name="c", subcore_axis_name="s")
    out_type = [jax.ShapeDtypeStruct((NC, N_PAD, D), jnp.float32)]
    if with_cnt:
        out_type.append(jax.ShapeDtypeStruct((NC * N_PAD,), jnp.float32))
    scratch = [
        pltpu.VMEM_SHARED((N_PAD, D), jnp.float32),
        pltpu.VMEM((CHUNK, D), jnp.float32),
        pltpu.VMEM((CHUNK, D), jnp.float32),
        pltpu.VMEM((CHUNK, D), jnp.float32),
        pltpu.VMEM((CHUNK, D), jnp.float32),
        pltpu.VMEM((BCH, CHUNK), jnp.int32),
        pltpu.VMEM((BCH, CHUNK), jnp.int32),
        pltpu.VMEM((BCH, CHUNK), jnp.float32),
        pltpu.SemaphoreType.DMA,
        pltpu.SemaphoreType.DMA,
        pltpu.SemaphoreType.DMA,
        pltpu.SemaphoreType.DMA,
        pltpu.SemaphoreType.DMA,
        pltpu.SemaphoreType.DMA,
        pltpu.SemaphoreType.DMA,
        pltpu.SemaphoreType.DMA,
    ]
    if with_cnt:
        scratch.extend([
            pltpu.VMEM_SHARED((N_PAD,), jnp.float32),
            pltpu.VMEM((ROWS_PER_TILE,), jnp.float32),
            pltpu.VMEM((CHUNK,), jnp.float32),
            pltpu.SemaphoreType.DMA,
        ])

    def body(src_hbm, dst_hbm, w_hbm, x_hbm, *rest):
        if with_cnt:
            (part_hbm, cnt_hbm, accum, rows0, rows1, rows2, rows3,
             srcv, dstv, wv,
             g0, g1, g2, g3, s0, s1, s2, s3,
             cnta, cntb, onev, scnt) = rest
        else:
            (part_hbm, accum, rows0, rows1, rows2, rows3,
             srcv, dstv, wv,
             g0, g1, g2, g3, s0, s1, s2, s3) = rest
            cnt_hbm = cnta = cntb = onev = scnt = None
        bufs = (rows0, rows1, rows2, rows3)
        gsems = (g0, g1, g2, g3)
        ssems = (s0, s1, s2, s3)
        c = lax.axis_index("c")
        s = lax.axis_index("s")
        wid = s * NC + c
        base = s * ROWS_PER_TILE
        zero = jnp.zeros((L,), jnp.float32)

        def zr(r, carry):
            for k in range(D // L):
                rows0[r, pl.ds(k * L, L)] = zero
            return carry
        lax.fori_loop(0, CHUNK, zr, 0)
        r0 = base
        for ln in _EPI:
            pltpu.sync_copy(rows0.at[pl.ds(0, ln)], accum.at[pl.ds(r0, ln)])
            r0 += ln

        if with_cnt:
            ones = jnp.ones((L,), jnp.float32)
            for k in range(CHUNK // L):
                onev[pl.ds(k * L, L)] = ones
            def zc(i, carry):
                cntb[pl.ds(i * L, L)] = zero
                return carry
            lax.fori_loop(0, ROWS_PER_TILE // L, zc, 0)
            if ROWS_PER_TILE % L:
                cntb[pl.ds(ROWS_PER_TILE - L, L)] = zero
            pltpu.sync_copy(cntb, cnta.at[pl.ds(base, ROWS_PER_TILE)])

        plsc.subcore_barrier()

        def issue_gather(jrow, buf, sem):
            pltpu.async_copy(x_hbm.at[srcv.at[jrow]], buf, sem)

        def wait_gather(buf, sem):
            pltpu.make_async_copy(x_hbm.at[srcv.at[0]], buf, sem).wait()

        def issue_scatter(jrow, buf, sem):
            pltpu.async_copy(buf, accum.at[dstv.at[jrow]], sem, add=True)

        def wait_scatter(buf, sem):
            pltpu.make_async_copy(buf, accum.at[dstv.at[0]], sem).wait()

        def issue_cnt(jrow):
            pltpu.async_copy(onev, cnta.at[dstv.at[jrow]], scnt, add=True)

        def wait_cnt():
            pltpu.make_async_copy(onev, cnta.at[dstv.at[0]], scnt).wait()

        def multiply(buf, jrow):
            def group_body(g, carry):
                wvec = wv[jrow, pl.ds(g * L, L)]
                for lane in range(L):
                    wspl = _dyn_splat(wvec, lane)
                    e = g * L + lane
                    for k in range(D // L):
                        buf[e, pl.ds(k * L, L)] = (
                            buf[e, pl.ds(k * L, L)] * wspl)
                return carry
            lax.fori_loop(0, CHUNK // L, group_body, 0)

        def block_body(b, carry):
            pltpu.sync_copy(src_hbm.at[wid, pl.ds(b * BCH, BCH)], srcv)
            pltpu.sync_copy(dst_hbm.at[wid, pl.ds(b * BCH, BCH)], dstv)
            pltpu.sync_copy(w_hbm.at[wid, pl.ds(b * BCH, BCH)], wv)
            issue_gather(0, bufs[0], gsems[0])
            issue_gather(1, bufs[1], gsems[1])

            def quad_body(t, carry1):
                for i in range(4):
                    c_blk = 4 * t + i
                    p = i
                    q = (i + 2) % 4
                    if i < 2:
                        @pl.when(t > 0)
                        def _(q=q):
                            wait_scatter(bufs[q], ssems[q])
                            if with_cnt:
                                wait_cnt()
                    else:
                        wait_scatter(bufs[q], ssems[q])
                        if with_cnt:
                            wait_cnt()
                    if i < 2:
                        issue_gather(c_blk + 2, bufs[q], gsems[q])
                    else:
                        @pl.when(t < BCH // 4 - 1)
                        def _(c_blk=c_blk, q=q):
                            issue_gather(c_blk + 2, bufs[q], gsems[q])
                    wait_gather(bufs[p], gsems[p])
                    multiply(bufs[p], c_blk)
                    issue_scatter(c_blk, bufs[p], ssems[p])
                    if with_cnt:
                        issue_cnt(c_blk)
                return carry1
            lax.fori_loop(0, BCH // 4, quad_body, 0)
            wait_scatter(bufs[2], ssems[2])
            wait_scatter(bufs[3], ssems[3])
            if with_cnt:
                wait_cnt()
                wait_cnt()
            return carry
        lax.fori_loop(0, NBLK, block_body, 0)

        plsc.subcore_barrier()

        r0 = base
        for ln in _EPI:
            pltpu.sync_copy(accum.at[pl.ds(r0, ln)], rows0.at[pl.ds(0, ln)])
            pltpu.sync_copy(rows0.at[pl.ds(0, ln)],
                            part_hbm.at[c, pl.ds(r0, ln)])
            r0 += ln
        if with_cnt:
            pltpu.sync_copy(cnta.at[pl.ds(base, ROWS_PER_TILE)], cntb)
            off = pl.multiple_of(c * N_PAD + base, 8)
            pltpu.sync_copy(cntb, cnt_hbm.at[pl.ds(off, ROWS_PER_TILE)])

    return pl.kernel(
        body, out_type=tuple(out_type), mesh=mesh, scratch_types=scratch,
        compiler_params=pltpu.CompilerParams(needs_layout_passes=False))


_layer1 = _make_layer(True)
_layer2 = _make_layer(False)


def _leaky(y):
    return jnp.where(y >= 0, y, 0.01 * y)


def _combine1_body(p_ref, c_ref, o_ref):
    acc = p_ref[0] + p_ref[1]
    cnt = jnp.sum(c_ref[...], axis=0)
    o_ref[...] = _leaky(acc / jnp.maximum(cnt, 1.0)[:, None])


def _combine1(part, cnt32):
    return pl.pallas_call(
        _combine1_body,
        out_shape=jax.ShapeDtypeStruct((N_PAD, D), jnp.float32),
        grid=(N_PAD // ROW_BLK,),
        in_specs=[
            pl.BlockSpec((NC, ROW_BLK, D), lambda i: (0, i, 0)),
            pl.BlockSpec((NC, ROW_BLK), lambda i: (0, i)),
        ],
        out_specs=pl.BlockSpec((ROW_BLK, D), lambda i: (i, 0)),
    )(part, cnt32)


def _combine2_body(p_ref, c_ref, x_ref, x1_ref, o_ref):
    acc = p_ref[0] + p_ref[1]
    cnt = jnp.sum(c_ref[...], axis=0)
    x2 = _leaky(acc / jnp.maximum(cnt, 1.0)[:, None])
    o_ref[...] = x_ref[...] + x1_ref[...] + x2


def _combine2(part, cnt32, x, x1):
    return pl.pallas_call(
        _combine2_body,
        out_shape=jax.ShapeDtypeStruct((N_PAD, D), jnp.float32),
        grid=(N_PAD // ROW_BLK,),
        in_specs=[
            pl.BlockSpec((NC, ROW_BLK, D), lambda i: (0, i, 0)),
            pl.BlockSpec((NC, ROW_BLK), lambda i: (0, i)),
            pl.BlockSpec((ROW_BLK, D), lambda i: (i, 0)),
            pl.BlockSpec((ROW_BLK, D), lambda i: (i, 0)),
        ],
        out_specs=pl.BlockSpec((ROW_BLK, D), lambda i: (i, 0)),
    )(part, cnt32, x, x1)


def kernel(edge_index, weight_vector, id_embedding):
    src = jnp.concatenate([edge_index[0], edge_index[1]])
    dst = jnp.concatenate([edge_index[1], edge_index[0]])
    w = weight_vector[:, 0]

    pad = E_PAD - E2
    ar = jnp.arange(pad, dtype=jnp.int32)
    src_p = jnp.concatenate([src, (ar * 97) % N_PAD])
    dst_p = jnp.concatenate([dst, N + (ar % (N_PAD - N))])
    w_p = jnp.concatenate([w, jnp.zeros((pad,), jnp.float32)])
    src_r = src_p.reshape(NW, CH_PER_W, CHUNK)
    dst_r = dst_p.reshape(NW, CH_PER_W, CHUNK)
    w_r = w_p.reshape(NW, CH_PER_W, CHUNK)

    x_pad = jnp.pad(id_embedding, ((0, N_PAD - N), (0, 0)))
    x = _l2norm(x_pad)

    part1, cnt_flat = _layer1(src_r, dst_r, w_r, x)
    cnt32 = cnt_flat.reshape(NC, N_PAD)
    x1 = _combine1(part1, cnt32)

    (part2,) = _layer2(src_r, dst_r, w_r, x1)
    out = _combine2(part2, cnt32, x, x1)
    return out[:N]
